# baseline (speedup 1.0000x reference)
import functools

import jax
import jax.numpy as jnp
from jax import lax
from jax.experimental import pallas as pl
from jax.experimental.pallas import tpu as pltpu
from jax.experimental.pallas import tpu_sc as plsc

BATCH = 16384
HID = 64
NC = 2
NS = 16
NW = NC * NS
BPW = BATCH // NW
L = 16
CH = 64
NB = CH // L
NCH = BPW // CH
BLK = 2048


def _transpose_body(x_ref, eye_ref, o_ref):
    xb = x_ref[...].astype(jnp.bfloat16)
    o_ref[...] = lax.dot_general(
        xb, eye_ref[...], (((0,), (0,)), ((), ())),
        preferred_element_type=jnp.float32).astype(jnp.bfloat16)


def _transpose_call(x_t, rows):
    grid = ((rows + BLK - 1) // BLK,)
    eye64 = jnp.eye(64, dtype=jnp.bfloat16)
    return pl.pallas_call(
        _transpose_body,
        grid=grid,
        in_specs=[pl.BlockSpec((64, BLK), lambda i: (0, i)),
                  pl.BlockSpec((64, 64), lambda i: (0, 0))],
        out_specs=pl.BlockSpec((BLK, 64), lambda i: (i, 0)),
        out_shape=jax.ShapeDtypeStruct((rows, 64), jnp.bfloat16),
    )(x_t, eye64)


def _make_sc_gather():
    mesh = plsc.VectorSubcoreMesh(core_axis_name="c", subcore_axis_name="s")

    @functools.partial(
        pl.kernel,
        out_type=[
            jax.ShapeDtypeStruct((BATCH, 8, HID), jnp.bfloat16),
            jax.ShapeDtypeStruct((BATCH, 8, HID), jnp.bfloat16),
        ],
        mesh=mesh,
        scratch_types=[
            pltpu.VMEM((BPW,), jnp.int32),
            pltpu.VMEM((BPW,), jnp.int32),
            pltpu.VMEM((CH, 8, HID), jnp.bfloat16),
            pltpu.VMEM((CH, 8, HID), jnp.bfloat16),
            pltpu.SemaphoreType.DMA,
        ],
        compiler_params=pltpu.CompilerParams(use_tc_tiling_on_sc=True),
    )
    def _sc_gather(sidx_hbm, qidx_hbm, stab_hbm, qtab_hbm, s_out, q_out,
                   sidx_v, qidx_v, srows_v, qrows_v, sem):
        wid = lax.axis_index("s") * NC + lax.axis_index("c")
        base = wid * BPW
        pltpu.sync_copy(sidx_hbm.at[pl.ds(base, BPW)], sidx_v)
        pltpu.sync_copy(qidx_hbm.at[pl.ds(base, BPW)], qidx_v)

        def chunk_body(c, _):
            copies = []
            for b in range(NB):
                sv = sidx_v[pl.ds(c * CH + b * L, L)]
                qv = qidx_v[pl.ds(c * CH + b * L, L)]
                for j in range(L):
                    ci = b * L + j
                    r8 = pl.multiple_of((sv[j] >> 3) << 3, 8)
                    copies.append(pltpu.async_copy(
                        stab_hbm.at[pl.ds(r8, 8)], srows_v.at[ci], sem))
                    rq8 = pl.multiple_of((qv[j] >> 3) << 3, 8)
                    copies.append(pltpu.async_copy(
                        qtab_hbm.at[pl.ds(rq8, 8)], qrows_v.at[ci], sem))
            for cp in copies:
                cp.wait()
            flush = pl.multiple_of(base + c * CH, CH)
            pltpu.sync_copy(srows_v, s_out.at[pl.ds(flush, CH)])
            pltpu.sync_copy(qrows_v, q_out.at[pl.ds(flush, CH)])
            return ()

        lax.fori_loop(0, NCH, chunk_body, (), unroll=False)

    return _sc_gather


BS = 1024


def _sel8(blocks, ids):
    b0 = (ids & 1) == 1
    b1 = (ids & 2) == 2
    b2 = (ids & 4) == 4
    s01 = jnp.where(b0, blocks[:, 1, :], blocks[:, 0, :])
    s23 = jnp.where(b0, blocks[:, 3, :], blocks[:, 2, :])
    s45 = jnp.where(b0, blocks[:, 5, :], blocks[:, 4, :])
    s67 = jnp.where(b0, blocks[:, 7, :], blocks[:, 6, :])
    s0123 = jnp.where(b1, s23, s01)
    s4567 = jnp.where(b1, s67, s45)
    return jnp.where(b2, s4567, s0123)


def _mlp_body(s_ref, q_ref, sid_ref, qid_ref, ts_ref, tt_ref, w1s_ref,
              w1q_ref, w1t_ref, b1_ref, w2_ref, b2_ref, w3_ref, b3_ref,
              out_ref):
    s = _sel8(s_ref[...], sid_ref[...]).astype(jnp.float32)
    q = _sel8(q_ref[...], qid_ref[...]).astype(jnp.float32)
    ts = ts_ref[...]
    binned = jnp.clip(ts // 60, 0, 9)
    oh = (binned == lax.broadcasted_iota(jnp.int32, (1, 16), 1)
          ).astype(jnp.float32)
    ttp = jnp.dot(tt_ref[...], w1t_ref[...],
                  preferred_element_type=jnp.float32)
    x1 = (jnp.dot(s, w1s_ref[...], preferred_element_type=jnp.float32)
          + jnp.dot(q, w1q_ref[...], preferred_element_type=jnp.float32)
          + jnp.dot(oh, ttp, preferred_element_type=jnp.float32)
          + b1_ref[...])
    h1 = jnp.maximum(x1, 0.0)
    h2 = jnp.maximum(
        jnp.dot(h1, w2_ref[...], preferred_element_type=jnp.float32)
        + b2_ref[...], 0.0)
    o = jnp.sum(h2 * w3_ref[...], axis=1, keepdims=True) + b3_ref[...]
    out_ref[...] = jax.nn.sigmoid(o)


def _mlp_call(s_g, q_g, sid2, qid2, ts2, tt16, w1s, w1q, w1t, b1r, w2, b2r,
              w3r, b3r):
    grid = (BATCH // BS,)
    return pl.pallas_call(
        _mlp_body,
        grid=grid,
        in_specs=[
            pl.BlockSpec((BS, 8, HID), lambda i: (i, 0, 0)),
            pl.BlockSpec((BS, 8, HID), lambda i: (i, 0, 0)),
            pl.BlockSpec((BS, 1), lambda i: (i, 0)),
            pl.BlockSpec((BS, 1), lambda i: (i, 0)),
            pl.BlockSpec((BS, 1), lambda i: (i, 0)),
            pl.BlockSpec((16, 32), lambda i: (0, 0)),
            pl.BlockSpec((HID, 128), lambda i: (0, 0)),
            pl.BlockSpec((HID, 128), lambda i: (0, 0)),
            pl.BlockSpec((32, 128), lambda i: (0, 0)),
            pl.BlockSpec((1, 128), lambda i: (0, 0)),
            pl.BlockSpec((128, HID), lambda i: (0, 0)),
            pl.BlockSpec((1, HID), lambda i: (0, 0)),
            pl.BlockSpec((1, HID), lambda i: (0, 0)),
            pl.BlockSpec((1, 1), lambda i: (0, 0)),
        ],
        out_specs=pl.BlockSpec((BS, 1), lambda i: (i, 0)),
        out_shape=jax.ShapeDtypeStruct((BATCH, 1), jnp.float32),
    )(s_g, q_g, sid2, qid2, ts2, tt16, w1s, w1q, w1t, b1r, w2, b2r, w3r, b3r)


def kernel(s_ids, i_ids, time_spent, student_table, item_table, time_table,
           W1, b1, W2, b2, W3, b3):
    stab_bf = _transpose_call(student_table.T, 1000000)
    qtab_bf = _transpose_call(item_table.T, 100000)
    s_g, q_g = _make_sc_gather()(s_ids, i_ids, stab_bf, qtab_bf)

    tt16 = jnp.zeros((16, 32), jnp.float32).at[:10].set(time_table)
    out = _mlp_call(
        s_g, q_g,
        s_ids.reshape(BATCH, 1),
        i_ids.reshape(BATCH, 1),
        time_spent.reshape(BATCH, 1),
        tt16,
        W1[:HID], W1[HID:2 * HID], W1[2 * HID:],
        b1.reshape(1, 128),
        W2,
        b2.reshape(1, HID),
        W3.reshape(1, HID),
        b3.reshape(1, 1),
    )
    return out


# speedup vs baseline: 1.4557x; 1.4557x over previous
import functools

import jax
import jax.numpy as jnp
from jax import lax
from jax.experimental import pallas as pl
from jax.experimental.pallas import tpu as pltpu
from jax.experimental.pallas import tpu_sc as plsc

BATCH = 16384
HID = 64
NC = 2
NS = 16
NW = NC * NS
BPW = BATCH // NW
K = 16
CH = 128
NBATCH = CH // K
NCH = BPW // CH


def _make_sc_gather():
    mesh = plsc.VectorSubcoreMesh(core_axis_name="c", subcore_axis_name="s")

    @functools.partial(
        pl.kernel,
        out_type=[
            jax.ShapeDtypeStruct((BATCH, HID), jnp.float32),
            jax.ShapeDtypeStruct((BATCH, HID), jnp.float32),
        ],
        mesh=mesh,
        scratch_types=[
            pltpu.VMEM((BPW,), jnp.int32),
            pltpu.VMEM((BPW,), jnp.int32),
            pltpu.VMEM((K, 8, HID), jnp.float32),
            pltpu.VMEM((K, 8, HID), jnp.float32),
            pltpu.VMEM((CH, HID), jnp.float32),
            pltpu.VMEM((CH, HID), jnp.float32),
            pltpu.SemaphoreType.DMA,
        ],
        compiler_params=pltpu.CompilerParams(use_tc_tiling_on_sc=True),
    )
    def _sc_gather(sidx_hbm, qidx_hbm, stab_hbm, qtab_hbm, s_out, q_out,
                   sidx_v, qidx_v, sstage_v, qstage_v, srows_v, qrows_v, sem):
        wid = lax.axis_index("s") * NC + lax.axis_index("c")
        base = wid * BPW
        pltpu.sync_copy(sidx_hbm.at[pl.ds(base, BPW)], sidx_v)
        pltpu.sync_copy(qidx_hbm.at[pl.ds(base, BPW)], qidx_v)

        def chunk_body(c, _):
            for b in range(NBATCH):
                off = c * CH + b * K
                sv = sidx_v[pl.ds(off, K)]
                qv = qidx_v[pl.ds(off, K)]
                copies = []
                srr = []
                qrr = []
                for j in range(K):
                    r = sv[j]
                    r8 = pl.multiple_of((r >> 3) << 3, 8)
                    srr.append(r & 7)
                    copies.append(pltpu.async_copy(
                        stab_hbm.at[pl.ds(r8, 8)], sstage_v.at[j], sem))
                    rq = qv[j]
                    rq8 = pl.multiple_of((rq >> 3) << 3, 8)
                    qrr.append(rq & 7)
                    copies.append(pltpu.async_copy(
                        qtab_hbm.at[pl.ds(rq8, 8)], qstage_v.at[j], sem))
                for cp in copies:
                    cp.wait()
                for j in range(K):
                    ci = b * K + j
                    for jj in range(HID // 16):
                        srows_v[ci, pl.ds(jj * 16, 16)] = (
                            sstage_v[j, srr[j], pl.ds(jj * 16, 16)])
                        qrows_v[ci, pl.ds(jj * 16, 16)] = (
                            qstage_v[j, qrr[j], pl.ds(jj * 16, 16)])
            flush = pl.multiple_of(base + c * CH, CH)
            pltpu.sync_copy(srows_v, s_out.at[pl.ds(flush, CH)])
            pltpu.sync_copy(qrows_v, q_out.at[pl.ds(flush, CH)])
            return ()

        lax.fori_loop(0, NCH, chunk_body, (), unroll=False)

    return _sc_gather


BS = 2048


def _mlp_body(s_ref, q_ref, ts_ref, tt_ref, w1s_ref, w1q_ref, w1t_ref,
              b1_ref, w2_ref, b2_ref, w3_ref, b3_ref, out_ref):
    s = s_ref[...]
    q = q_ref[...]
    ts = ts_ref[...]
    binned = jnp.clip(ts // 60, 0, 9)
    oh = (binned == lax.broadcasted_iota(jnp.int32, (1, 16), 1)
          ).astype(jnp.float32)
    ttp = jnp.dot(tt_ref[...], w1t_ref[...],
                  preferred_element_type=jnp.float32)
    x1 = (jnp.dot(s, w1s_ref[...], preferred_element_type=jnp.float32)
          + jnp.dot(q, w1q_ref[...], preferred_element_type=jnp.float32)
          + jnp.dot(oh, ttp, preferred_element_type=jnp.float32)
          + b1_ref[...])
    h1 = jnp.maximum(x1, 0.0)
    h2 = jnp.maximum(
        jnp.dot(h1, w2_ref[...], preferred_element_type=jnp.float32)
        + b2_ref[...], 0.0)
    o = jnp.sum(h2 * w3_ref[...], axis=1, keepdims=True) + b3_ref[...]
    out_ref[...] = jax.nn.sigmoid(o)


def _mlp_call(s_g, q_g, ts2, tt16, w1s, w1q, w1t, b1r, w2, b2r, w3r, b3r):
    grid = (BATCH // BS,)
    return pl.pallas_call(
        _mlp_body,
        grid=grid,
        in_specs=[
            pl.BlockSpec((BS, HID), lambda i: (i, 0)),
            pl.BlockSpec((BS, HID), lambda i: (i, 0)),
            pl.BlockSpec((BS, 1), lambda i: (i, 0)),
            pl.BlockSpec((16, 32), lambda i: (0, 0)),
            pl.BlockSpec((HID, 128), lambda i: (0, 0)),
            pl.BlockSpec((HID, 128), lambda i: (0, 0)),
            pl.BlockSpec((32, 128), lambda i: (0, 0)),
            pl.BlockSpec((1, 128), lambda i: (0, 0)),
            pl.BlockSpec((128, HID), lambda i: (0, 0)),
            pl.BlockSpec((1, HID), lambda i: (0, 0)),
            pl.BlockSpec((1, HID), lambda i: (0, 0)),
            pl.BlockSpec((1, 1), lambda i: (0, 0)),
        ],
        out_specs=pl.BlockSpec((BS, 1), lambda i: (i, 0)),
        out_shape=jax.ShapeDtypeStruct((BATCH, 1), jnp.float32),
    )(s_g, q_g, ts2, tt16, w1s, w1q, w1t, b1r, w2, b2r, w3r, b3r)


def kernel(s_ids, i_ids, time_spent, student_table, item_table, time_table,
           W1, b1, W2, b2, W3, b3):
    s_g, q_g = _make_sc_gather()(s_ids, i_ids, student_table, item_table)

    tt16 = jnp.zeros((16, 32), jnp.float32).at[:10].set(time_table)
    out = _mlp_call(
        s_g, q_g,
        time_spent.reshape(BATCH, 1),
        tt16,
        W1[:HID], W1[HID:2 * HID], W1[2 * HID:],
        b1.reshape(1, 128),
        W2,
        b2.reshape(1, HID),
        W3.reshape(1, HID),
        b3.reshape(1, 1),
    )
    return out
